# Initial kernel scaffold; baseline (speedup 1.0000x reference)
import jax
import jax.numpy as jnp
from jax.experimental import pallas as pl


def kernel(token_indices, embed_table):
    raise NotImplementedError("write your pallas kernel here")


# speedup vs baseline: 2.0547x; 2.0547x over previous
import functools
import math

import jax
import jax.numpy as jnp
from jax import lax
from jax.experimental import pallas as pl
from jax.experimental.pallas import tpu as pltpu
from jax.experimental.pallas import tpu_sc as plsc

D = 128
L = 16
NVEC = D // L

_SQRT_D = math.sqrt(float(D))


def _rsqrt(x):
    i = plsc.bitcast(x, jnp.int32)
    i = jnp.int32(0x5F3759DF) - (i >> 1)
    y = plsc.bitcast(i, jnp.float32)
    for _ in range(3):
        y = y * (1.5 - 0.5 * x * y * y)
    return y


_GATHER_DNUMS = lax.GatherDimensionNumbers(
    offset_dims=(), collapsed_slice_dims=(0,), start_index_map=(0,))


def _shuffle(v, idx):
    return lax.gather(v, idx[:, None], _GATHER_DNUMS, (1,),
                      mode=lax.GatherScatterMode.PROMISE_IN_BOUNDS)


def _make_kernel(total_rows):
    info = plsc.get_sparse_core_info()
    nc, ns = info.num_cores, info.num_subcores
    nw = nc * ns
    per_w = total_rows // nw
    chunk = 128
    steps = per_w // chunk

    mesh = plsc.VectorSubcoreMesh(core_axis_name="c", subcore_axis_name="s")

    @functools.partial(
        pl.kernel,
        mesh=mesh,
        compiler_params=pltpu.CompilerParams(needs_layout_passes=False),
        out_type=jax.ShapeDtypeStruct((total_rows, D), jnp.float32),
        scratch_types=[
            pltpu.VMEM((chunk,), jnp.int32),
            pltpu.VMEM((chunk, D), jnp.float32),
            pltpu.SemaphoreType.DMA,
        ],
    )
    def emb_kernel(idx_hbm, table_hbm, out_hbm, idx_v, rows_v, sem):
        cid = lax.axis_index("c")
        sid = lax.axis_index("s")
        wid = sid * nc + cid
        base = wid * per_w
        lanes = lax.iota(jnp.int32, L)
        perms = [lanes ^ (1 << k) for k in range(4)]

        def step(g, carry):
            off = base + g * chunk
            pltpu.sync_copy(idx_hbm.at[pl.ds(off, chunk)], idx_v)
            pltpu.async_copy(table_hbm.at[idx_v], rows_v, sem).wait()

            def row(r, carry2):
                xs = [rows_v[r, pl.ds(k * L, L)] for k in range(NVEC)]
                acc = xs[0] * xs[0]
                for k in range(1, NVEC):
                    acc = acc + xs[k] * xs[k]
                for p in perms:
                    acc = acc + _shuffle(acc, p)
                scale = _rsqrt(jnp.maximum(acc, 1e-24)) * _SQRT_D
                for k in range(NVEC):
                    rows_v[r, pl.ds(k * L, L)] = xs[k] * scale
                return carry2

            lax.fori_loop(0, chunk, row, 0)
            pltpu.sync_copy(rows_v, out_hbm.at[pl.ds(off, chunk)])
            return carry

        lax.fori_loop(0, steps, step, 0)

    return emb_kernel


def kernel(token_indices, embed_table):
    b, h = token_indices.shape
    total = b * h
    idx_flat = token_indices.reshape(total).astype(jnp.int32)
    out = _make_kernel(total)(idx_flat, embed_table)
    return out.reshape(b, h, embed_table.shape[1])


# speedup vs baseline: 6.4203x; 3.1247x over previous
import functools
import math

import jax
import jax.numpy as jnp
from jax import lax
from jax.experimental import pallas as pl
from jax.experimental.pallas import tpu as pltpu
from jax.experimental.pallas import tpu_sc as plsc

D = 128
L = 16
NVEC = D // L
CH = 128
NBUF = 2

_SQRT_D = math.sqrt(float(D))

_GATHER_DNUMS = lax.GatherDimensionNumbers(
    offset_dims=(), collapsed_slice_dims=(0,), start_index_map=(0,))


def _shuffle(v, idx):
    return lax.gather(v, idx[:, None], _GATHER_DNUMS, (1,),
                      mode=lax.GatherScatterMode.PROMISE_IN_BOUNDS)


def _rsqrt(x):
    i = plsc.bitcast(x, jnp.int32)
    i = jnp.int32(0x5F3759DF) - (i >> 1)
    y = plsc.bitcast(i, jnp.float32)
    xh = 0.5 * x
    for _ in range(2):
        y = y * (1.5 - xh * y * y)
    return y


def _make_kernel(total_rows):
    info = plsc.get_sparse_core_info()
    nc, ns = info.num_cores, info.num_subcores
    nw = nc * ns
    per_w = total_rows // nw
    steps = per_w // CH
    njs = steps // NBUF

    mesh = plsc.VectorSubcoreMesh(core_axis_name="c", subcore_axis_name="s")

    @functools.partial(
        pl.kernel,
        mesh=mesh,
        compiler_params=pltpu.CompilerParams(needs_layout_passes=False),
        out_type=jax.ShapeDtypeStruct((total_rows, D), jnp.float32),
        scratch_types=[
            pltpu.VMEM((NBUF, CH), jnp.int32),
            pltpu.VMEM((NBUF, CH, D), jnp.float32),
            pltpu.VMEM((NBUF, CH, D), jnp.float32),
            pltpu.SemaphoreType.DMA,
            pltpu.SemaphoreType.DMA,
            pltpu.SemaphoreType.DMA,
            pltpu.SemaphoreType.DMA,
        ],
    )
    def emb_kernel(idx_hbm, table_hbm, out_hbm, idx_v, rin, rout,
                   g0, g1, s0, s1):
        cid = lax.axis_index("c")
        sid = lax.axis_index("s")
        wid = sid * nc + cid
        base = wid * per_w
        gsems = (g0, g1)
        ssems = (s0, s1)
        lanes = lax.iota(jnp.int32, L)
        perms = [lanes ^ (1 << k) for k in range(4)]

        def gather_chunk(b, off):
            pltpu.sync_copy(idx_hbm.at[pl.ds(off, CH)], idx_v.at[b])
            pltpu.make_async_copy(
                table_hbm.at[idx_v.at[b]], rin.at[b], gsems[b]).start()

        def compute_chunk(b):
            rin_b = rin.at[b]
            rout_b = rout.at[b]

            @plsc.parallel_loop(0, CH, unroll=2)
            def row(r):
                xs = [rin_b[r, pl.ds(k * L, L)] for k in range(NVEC)]
                sq = [x * x for x in xs]
                for st in (4, 2, 1):
                    sq = [sq[k] + sq[k + st] for k in range(st)]
                acc = sq[0]
                for p in perms:
                    acc = acc + _shuffle(acc, p)
                scale = _rsqrt(jnp.maximum(acc, 1e-24)) * _SQRT_D
                for k in range(NVEC):
                    rout_b[r, pl.ds(k * L, L)] = xs[k] * scale

        for b in range(NBUF):
            gather_chunk(b, base + b * CH)

        def trip(j, carry):
            for b in range(NBUF):
                i = j * NBUF + b
                off = base + i * CH
                pltpu.make_async_copy(
                    table_hbm.at[idx_v.at[b]], rin.at[b], gsems[b]).wait()

                @pl.when(j > 0)
                def _():
                    pltpu.make_async_copy(
                        rout.at[b], out_hbm.at[pl.ds(off, CH)],
                        ssems[b]).wait()

                compute_chunk(b)
                pltpu.make_async_copy(
                    rout.at[b], out_hbm.at[pl.ds(off, CH)], ssems[b]).start()

                @pl.when(j < njs - 1)
                def _():
                    gather_chunk(b, off + NBUF * CH)
            return carry

        lax.fori_loop(0, njs, trip, 0)

        for b in range(NBUF):
            off = base + (steps - NBUF + b) * CH
            pltpu.make_async_copy(
                rout.at[b], out_hbm.at[pl.ds(off, CH)], ssems[b]).wait()

    return emb_kernel


def kernel(token_indices, embed_table):
    b, h = token_indices.shape
    total = b * h
    idx_flat = token_indices.reshape(total).astype(jnp.int32)
    out = _make_kernel(total)(idx_flat, embed_table)
    return out.reshape(b, h, embed_table.shape[1])


# speedup vs baseline: 8.2246x; 1.2810x over previous
import functools
import math

import jax
import jax.numpy as jnp
from jax import lax
from jax.experimental import pallas as pl
from jax.experimental.pallas import tpu as pltpu
from jax.experimental.pallas import tpu_sc as plsc

D = 128
L = 16
NVEC = D // L
CH = 128
NBUF = 2

_SQRT_D = math.sqrt(float(D))

_GATHER_DNUMS = lax.GatherDimensionNumbers(
    offset_dims=(), collapsed_slice_dims=(0,), start_index_map=(0,))


def _shuffle(v, idx):
    return lax.gather(v, idx[:, None], _GATHER_DNUMS, (1,),
                      mode=lax.GatherScatterMode.PROMISE_IN_BOUNDS)


def _rsqrt(x):
    i = plsc.bitcast(x, jnp.int32)
    i = jnp.int32(0x5F3759DF) - (i >> 1)
    y = plsc.bitcast(i, jnp.float32)
    xh = 0.5 * x
    y = y * (1.5 - xh * y * y)
    return y


def _make_kernel(total_rows):
    info = plsc.get_sparse_core_info()
    nc, ns = info.num_cores, info.num_subcores
    nw = nc * ns
    per_w = total_rows // nw
    steps = per_w // CH
    njs = steps // NBUF

    mesh = plsc.VectorSubcoreMesh(core_axis_name="c", subcore_axis_name="s")

    @functools.partial(
        pl.kernel,
        mesh=mesh,
        compiler_params=pltpu.CompilerParams(needs_layout_passes=False),
        out_type=jax.ShapeDtypeStruct((total_rows, D), jnp.float32),
        scratch_types=[
            pltpu.VMEM((NBUF, CH), jnp.int32),
            pltpu.VMEM((NBUF, CH, D), jnp.float32),
            pltpu.VMEM((NBUF, CH, D), jnp.float32),
            pltpu.SemaphoreType.DMA,
            pltpu.SemaphoreType.DMA,
            pltpu.SemaphoreType.DMA,
            pltpu.SemaphoreType.DMA,
        ],
    )
    def emb_kernel(idx_hbm, table_hbm, out_hbm, idx_v, rin, rout,
                   g0, g1, s0, s1):
        cid = lax.axis_index("c")
        sid = lax.axis_index("s")
        wid = sid * nc + cid
        base = wid * per_w
        gsems = (g0, g1)
        ssems = (s0, s1)
        lanes = lax.iota(jnp.int32, L)
        perms = [lanes ^ (1 << k) for k in range(4)]

        def gather_chunk(b, off):
            pltpu.sync_copy(idx_hbm.at[pl.ds(off, CH)], idx_v.at[b])
            pltpu.make_async_copy(
                table_hbm.at[idx_v.at[b]], rin.at[b], gsems[b]).start()

        def compute_chunk(b):
            rin_b = rin.at[b]
            rout_b = rout.at[b]

            @plsc.parallel_loop(0, CH, unroll=4)
            def row(r):
                xs = [rin_b[r, pl.ds(k * L, L)] for k in range(NVEC)]
                sq = [x * x for x in xs]
                for st in (4, 2, 1):
                    sq = [sq[k] + sq[k + st] for k in range(st)]
                acc = sq[0]
                for p in perms:
                    acc = acc + _shuffle(acc, p)
                scale = _rsqrt(jnp.maximum(acc, 1e-24)) * _SQRT_D
                for k in range(NVEC):
                    rout_b[r, pl.ds(k * L, L)] = xs[k] * scale

        for b in range(NBUF):
            gather_chunk(b, base + b * CH)

        def trip(j, carry):
            for b in range(NBUF):
                i = j * NBUF + b
                off = base + i * CH
                pltpu.make_async_copy(
                    table_hbm.at[idx_v.at[b]], rin.at[b], gsems[b]).wait()

                @pl.when(j > 0)
                def _():
                    pltpu.make_async_copy(
                        rout.at[b], out_hbm.at[pl.ds(off, CH)],
                        ssems[b]).wait()

                compute_chunk(b)
                pltpu.make_async_copy(
                    rout.at[b], out_hbm.at[pl.ds(off, CH)], ssems[b]).start()

                @pl.when(j < njs - 1)
                def _():
                    gather_chunk(b, off + NBUF * CH)
            return carry

        lax.fori_loop(0, njs, trip, 0)

        for b in range(NBUF):
            off = base + (steps - NBUF + b) * CH
            pltpu.make_async_copy(
                rout.at[b], out_hbm.at[pl.ds(off, CH)], ssems[b]).wait()

    return emb_kernel


def kernel(token_indices, embed_table):
    b, h = token_indices.shape
    total = b * h
    idx_flat = token_indices.reshape(total).astype(jnp.int32)
    out = _make_kernel(total)(idx_flat, embed_table)
    return out.reshape(b, h, embed_table.shape[1])


# speedup vs baseline: 9.5830x; 1.1652x over previous
import functools
import math

import jax
import jax.numpy as jnp
from jax import lax
from jax.experimental import pallas as pl
from jax.experimental.pallas import tpu as pltpu
from jax.experimental.pallas import tpu_sc as plsc

D = 128
L = 16
NVEC = D // L
CH = 128
NBUF = 2

_SQRT_D = math.sqrt(float(D))

_GATHER_DNUMS = lax.GatherDimensionNumbers(
    offset_dims=(), collapsed_slice_dims=(0,), start_index_map=(0,))


def _shuffle(v, idx):
    return lax.gather(v, idx[:, None], _GATHER_DNUMS, (1,),
                      mode=lax.GatherScatterMode.PROMISE_IN_BOUNDS)


def _scale_from_sumsq(n2):
    a = jnp.maximum(n2, 1.28e-22)
    i = jnp.int32(0x5F3759DF + (7 << 22)) - (plsc.bitcast(a, jnp.int32) >> 1)
    y = plsc.bitcast(i, jnp.float32)
    xh = a * (0.5 / D)
    y = y * (1.5 - xh * y * y)
    return y


def _make_kernel(total_rows):
    info = plsc.get_sparse_core_info()
    nc, ns = info.num_cores, info.num_subcores
    nw = nc * ns
    per_w = total_rows // nw
    steps = per_w // CH
    njs = steps // NBUF

    mesh = plsc.VectorSubcoreMesh(core_axis_name="c", subcore_axis_name="s")

    @functools.partial(
        pl.kernel,
        mesh=mesh,
        compiler_params=pltpu.CompilerParams(needs_layout_passes=False),
        out_type=jax.ShapeDtypeStruct((total_rows, D), jnp.float32),
        scratch_types=[
            pltpu.VMEM((per_w,), jnp.int32),
            pltpu.VMEM((NBUF, CH, D), jnp.float32),
            pltpu.VMEM((NBUF, CH, D), jnp.float32),
            pltpu.SemaphoreType.DMA,
            pltpu.SemaphoreType.DMA,
            pltpu.SemaphoreType.DMA,
            pltpu.SemaphoreType.DMA,
        ],
    )
    def emb_kernel(idx_hbm, table_hbm, out_hbm, idx_v, rin, rout,
                   g0, g1, s0, s1):
        cid = lax.axis_index("c")
        sid = lax.axis_index("s")
        wid = sid * nc + cid
        base = wid * per_w
        gsems = (g0, g1)
        ssems = (s0, s1)
        last_lane = jnp.full((L,), L - 1, jnp.int32)

        pltpu.sync_copy(idx_hbm.at[pl.ds(base, per_w)], idx_v)

        def gather_chunk(b, i):
            pltpu.make_async_copy(
                table_hbm.at[idx_v.at[pl.ds(i * CH, CH)]],
                rin.at[b], gsems[b]).start()

        def compute_chunk(b):
            rin_b = rin.at[b]
            rout_b = rout.at[b]

            @plsc.parallel_loop(0, CH, unroll=4)
            def row(r):
                xs = [rin_b[r, pl.ds(k * L, L)] for k in range(NVEC)]
                sq = [x * x for x in xs]
                for st in (4, 2, 1):
                    sq = [sq[k] + sq[k + st] for k in range(st)]
                c = jnp.cumsum(sq[0])
                acc = _shuffle(c, last_lane)
                scale = _scale_from_sumsq(acc)
                for k in range(NVEC):
                    rout_b[r, pl.ds(k * L, L)] = xs[k] * scale

        for b in range(NBUF):
            gather_chunk(b, b)

        def trip(j, carry):
            for b in range(NBUF):
                i = j * NBUF + b
                off = base + i * CH
                pltpu.make_async_copy(
                    table_hbm.at[idx_v.at[pl.ds(i * CH, CH)]],
                    rin.at[b], gsems[b]).wait()

                @pl.when(j > 0)
                def _():
                    pltpu.make_async_copy(
                        rout.at[b], out_hbm.at[pl.ds(off, CH)],
                        ssems[b]).wait()

                compute_chunk(b)
                pltpu.make_async_copy(
                    rout.at[b], out_hbm.at[pl.ds(off, CH)], ssems[b]).start()

                @pl.when(j < njs - 1)
                def _():
                    gather_chunk(b, i + NBUF)
            return carry

        lax.fori_loop(0, njs, trip, 0)

        for b in range(NBUF):
            off = base + (steps - NBUF + b) * CH
            pltpu.make_async_copy(
                rout.at[b], out_hbm.at[pl.ds(off, CH)], ssems[b]).wait()

    return emb_kernel


def kernel(token_indices, embed_table):
    b, h = token_indices.shape
    total = b * h
    idx_flat = token_indices.reshape(total).astype(jnp.int32)
    out = _make_kernel(total)(idx_flat, embed_table)
    return out.reshape(b, h, embed_table.shape[1])
